# Initial kernel scaffold; baseline (speedup 1.0000x reference)
import jax
import jax.numpy as jnp
from jax.experimental import pallas as pl


def kernel(h_atom, h_query, edge_index, edge_attr, n_query, WQ, WK, WV, Wrbf, W1, b1, W2, b2, ln_gamma, ln_beta):
    raise NotImplementedError("write your pallas kernel here")


# speedup vs baseline: 1.4270x; 1.4270x over previous
import functools

import jax
import jax.numpy as jnp
from jax import lax
from jax.experimental import pallas as pl
from jax.experimental.pallas import tpu as pltpu
from jax.experimental.pallas import tpu_sc as plsc

N_ATOM = 10000
HID = 128
H = 4
D = HID // H
INV_SQRT_D = 1.0 / (D ** 0.5)

NQP = 10240
TRASH = 10000
NW = 32
CH = 64
BLK = 8
CPW = 160
NQD = NQP // 32
ROWS_PER_TILE = NQP // 16
DEN_ROWS_PER_TILE = NQD // 8


def _proj_body(hq_ref, ha_ref, wqt_ref, wkt_ref, wvt_ref, qq_ref, ka_ref, va_ref):
    qq_ref[...] = jnp.dot(hq_ref[...], wqt_ref[...], preferred_element_type=jnp.float32)
    ka_ref[...] = jnp.dot(ha_ref[...], wkt_ref[...], preferred_element_type=jnp.float32)
    va_ref[...] = jnp.dot(ha_ref[...], wvt_ref[...], preferred_element_type=jnp.float32)


def _proj(hq_pad, ha_pad, wqt, wkt, wvt):
    n = hq_pad.shape[0]
    blk = 512
    grid = n // blk
    rowspec = pl.BlockSpec((blk, HID), lambda i: (i, 0))
    wspec = pl.BlockSpec((HID, HID), lambda i: (0, 0))
    return pl.pallas_call(
        _proj_body,
        grid=(grid,),
        in_specs=[rowspec, rowspec, wspec, wspec, wspec],
        out_specs=[rowspec, rowspec, rowspec],
        out_shape=[jax.ShapeDtypeStruct((n, HID), jnp.float32)] * 3,
    )(hq_pad, ha_pad, wqt, wkt, wvt)


def _rbf_body(ea_ref, w_ref, out_ref):
    out_ref[...] = jnp.dot(ea_ref[...], w_ref[...], preferred_element_type=jnp.float32)


def _rbf(ea_pad, wrbf_pad_t):
    e = ea_pad.shape[0]
    blk = 2048
    grid = e // blk
    return pl.pallas_call(
        _rbf_body,
        grid=(grid,),
        in_specs=[pl.BlockSpec((blk, 16), lambda i: (i, 0)),
                  pl.BlockSpec((16, 16), lambda i: (0, 0))],
        out_specs=pl.BlockSpec((blk, 16), lambda i: (i, 0)),
        out_shape=jax.ShapeDtypeStruct((e, 16), jnp.float32),
    )(ea_pad, wrbf_pad_t)


def _edge_call(qq, ka, va, src2d, dst2d, rbf):
    n_rows = src2d.shape[0]
    cpw = n_rows // NW
    mesh = plsc.VectorSubcoreMesh(core_axis_name="c", subcore_axis_name="s")

    @functools.partial(
        pl.kernel,
        mesh=mesh,
        compiler_params=pltpu.CompilerParams(needs_layout_passes=False),
        out_type=[jax.ShapeDtypeStruct((2 * NQP, HID), jnp.float32),
                  jax.ShapeDtypeStruct((2 * NQD, HID), jnp.float32)],
        scratch_types=[
            pltpu.VMEM((BLK, CH), jnp.int32),
            pltpu.VMEM((BLK, CH), jnp.int32),
            pltpu.VMEM((CH,), jnp.int32),
            pltpu.VMEM((CH,), jnp.int32),
            pltpu.VMEM((CH,), jnp.int32),
            pltpu.VMEM((CH,), jnp.int32),
            pltpu.VMEM((CH, HID), jnp.float32),
            pltpu.VMEM((CH, HID), jnp.float32),
            pltpu.VMEM((CH, HID), jnp.float32),
            pltpu.VMEM((CH, 16), jnp.float32),
            pltpu.VMEM((CH, HID), jnp.float32),
            pltpu.VMEM_SHARED((NQP, HID), jnp.float32),
            pltpu.VMEM_SHARED((NQD, HID), jnp.float32),
            pltpu.SemaphoreType.DMA,
            pltpu.SemaphoreType.DMA,
            pltpu.SemaphoreType.DMA,
        ],
    )
    def edge_kernel(qq_hbm, ka_hbm, va_hbm, src2_hbm, dst2_hbm, rbf_hbm,
                    num_out, den_out,
                    src_blk, dst_blk, src_v, dst_v, dst_sc, dsh_v,
                    q_rows, k_rows, v_rows, rbf_v, den_src,
                    num_sh, den_sh, gqk, grbf, gv):
        cid = lax.axis_index("c")
        sid = lax.axis_index("s")
        wid = cid * 16 + sid
        zeros16 = jnp.zeros((16,), jnp.float32)
        row0_w = wid * cpw

        def zero_row(i, _):
            for j in range(HID // 16):
                v_rows[i, pl.ds(j * 16, 16)] = zeros16
                den_src[i, pl.ds(j * 16, 16)] = zeros16
            return 0
        lax.fori_loop(0, CH, zero_row, 0)
        for t in range(ROWS_PER_TILE // CH):
            r0 = sid * ROWS_PER_TILE + t * CH
            pltpu.sync_copy(v_rows, num_sh.at[pl.ds(r0, CH)])
        @pl.when(sid < 8)
        def _():
            pltpu.sync_copy(den_src.at[pl.ds(0, DEN_ROWS_PER_TILE)],
                            den_sh.at[pl.ds(sid * DEN_ROWS_PER_TILE, DEN_ROWS_PER_TILE)])
        plsc.subcore_barrier()

        def copy_idx(blk_ref, dst_ref, bi):
            for t in range(CH // 16):
                dst_ref[pl.ds(t * 16, 16)] = blk_ref[bi, pl.ds(t * 16, 16)]

        def load_blocks(b0):
            r = pl.multiple_of(row0_w + b0, 8)
            pltpu.sync_copy(src2_hbm.at[pl.ds(r, BLK)], src_blk)
            pltpu.sync_copy(dst2_hbm.at[pl.ds(r, BLK)], dst_blk)

        def fire(c):
            base = (row0_w + c) * CH
            pltpu.async_copy(rbf_hbm.at[pl.ds(base, CH)], rbf_v, grbf)
            pltpu.async_copy(qq_hbm.at[dst_v], q_rows, gqk)
            pltpu.async_copy(ka_hbm.at[src_v], k_rows, gqk)

        load_blocks(0)
        copy_idx(src_blk, src_v, 0)
        copy_idx(dst_blk, dst_v, 0)
        fire(0)
        pltpu.async_copy(va_hbm.at[src_v], v_rows, gv)

        def chunk_body(c, _):
            bi = lax.rem(c, BLK)
            copy_idx(dst_blk, dst_sc, bi)

            pltpu.make_async_copy(rbf_hbm.at[pl.ds(0, CH)], rbf_v, grbf).wait()
            pltpu.make_async_copy(qq_hbm.at[dst_v], q_rows, gqk).wait()
            pltpu.make_async_copy(ka_hbm.at[src_v], k_rows, gqk).wait()

            def scores_body(g, _):
                eidx = g * 16 + lax.iota(jnp.int32, 16)
                dm = dst_sc[pl.ds(g * 16, 16)]
                dsh_v[pl.ds(g * 16, 16)] = lax.shift_right_logical(dm, 5)
                dlow4 = (dm & 31) * 4
                for h in range(H):
                    hvec = jnp.full((16,), h, jnp.int32)
                    rb = plsc.load_gather(rbf_v, [eidx, hvec])
                    dot = jnp.zeros((16,), jnp.float32)
                    for j in range(D):
                        col = jnp.full((16,), h * D + j, jnp.int32)
                        qc = plsc.load_gather(q_rows, [eidx, col])
                        kc = plsc.load_gather(k_rows, [eidx, col])
                        dot = dot + qc * kc
                    ex = jnp.exp(dot * INV_SQRT_D + rb)
                    plsc.store_scatter(den_src, [eidx, dlow4 + h], ex)
                return 0
            lax.fori_loop(0, CH // 16, scores_body, 0)

            pltpu.make_async_copy(va_hbm.at[src_v], v_rows, gv).wait()

            def msg_body(g, _):
                eidx = g * 16 + lax.iota(jnp.int32, 16)
                dlow4 = (dst_sc[pl.ds(g * 16, 16)] & 31) * 4
                for h in range(H):
                    ex = plsc.load_gather(den_src, [eidx, dlow4 + h])
                    for j in range(D):
                        col = jnp.full((16,), h * D + j, jnp.int32)
                        vc = plsc.load_gather(v_rows, [eidx, col])
                        plsc.store_scatter(v_rows, [eidx, col], vc * ex)
                return 0
            lax.fori_loop(0, CH // 16, msg_body, 0)

            @pl.when(c + 1 < cpw)
            def _():
                @pl.when(lax.rem(c + 1, BLK) == 0)
                def _():
                    load_blocks(c + 1)
                bj = lax.rem(c + 1, BLK)
                copy_idx(src_blk, src_v, bj)
                copy_idx(dst_blk, dst_v, bj)
                fire(c + 1)

            pltpu.sync_copy(v_rows, num_sh.at[dst_sc], add=True)
            @pl.when(c + 1 < cpw)
            def _():
                pltpu.async_copy(va_hbm.at[src_v], v_rows, gv)
            pltpu.sync_copy(den_src, den_sh.at[dsh_v], add=True)

            def rezero_body(g, _):
                eidx = g * 16 + lax.iota(jnp.int32, 16)
                dlow4 = (dst_sc[pl.ds(g * 16, 16)] & 31) * 4
                for h in range(H):
                    plsc.store_scatter(den_src, [eidx, dlow4 + h], zeros16)
                return 0
            lax.fori_loop(0, CH // 16, rezero_body, 0)
            return 0
        lax.fori_loop(0, cpw, chunk_body, 0)

        plsc.subcore_barrier()
        rbase = sid * ROWS_PER_TILE
        pltpu.sync_copy(num_sh.at[pl.ds(rbase, ROWS_PER_TILE)],
                        num_out.at[pl.ds(cid * NQP + rbase, ROWS_PER_TILE)])
        @pl.when(sid < 8)
        def _():
            dbase = sid * DEN_ROWS_PER_TILE
            pltpu.sync_copy(den_sh.at[pl.ds(dbase, DEN_ROWS_PER_TILE)],
                            den_out.at[pl.ds(cid * NQD + dbase, DEN_ROWS_PER_TILE)])

    return edge_kernel(qq, ka, va, src2d, dst2d, rbf)


def _mlp_body(hq_ref, n0_ref, n1_ref, d0_ref, d1_ref,
              w1at_ref, w1bt_ref, w2t_ref, b1_ref, b2_ref, g_ref, be_ref,
              out_ref):
    den = d0_ref[...] + d1_ref[...]
    inv = 1.0 / (den + 1e-30)
    num = n0_ref[...] + n1_ref[...]
    r = num.shape[0]
    invw = jnp.concatenate(
        [jnp.broadcast_to(inv[:, h:h + 1], (r, D)) for h in range(H)], axis=1)
    agg = num * invw
    hq = hq_ref[...]
    hdn = jnp.dot(hq, w1at_ref[...], preferred_element_type=jnp.float32)
    hdn = hdn + jnp.dot(agg, w1bt_ref[...], preferred_element_type=jnp.float32)
    hdn = jnp.maximum(hdn + b1_ref[...], 0.0)
    delta = jnp.dot(hdn, w2t_ref[...], preferred_element_type=jnp.float32) + b2_ref[...]
    y = hq + delta
    mu = jnp.mean(y, axis=-1, keepdims=True)
    yc = y - mu
    var = jnp.mean(yc * yc, axis=-1, keepdims=True)
    out_ref[...] = yc * lax.rsqrt(var + 1e-5) * g_ref[...] + be_ref[...]


def _mlp(hq, n0, n1, d0, d1, w1at, w1bt, w2t, b1, b2, gamma, beta):
    n = hq.shape[0]
    blk = 1000
    grid = n // blk
    rowspec = pl.BlockSpec((blk, HID), lambda i: (i, 0))
    denspec = pl.BlockSpec((blk, 16), lambda i: (i, 0))
    wspec = pl.BlockSpec((HID, HID), lambda i: (0, 0))
    vspec = pl.BlockSpec((1, HID), lambda i: (0, 0))
    return pl.pallas_call(
        _mlp_body,
        grid=(grid,),
        in_specs=[rowspec, rowspec, rowspec, denspec, denspec,
                  wspec, wspec, wspec, vspec, vspec, vspec, vspec],
        out_specs=rowspec,
        out_shape=jax.ShapeDtypeStruct((n, HID), jnp.float32),
    )(hq, n0, n1, d0, d1, w1at, w1bt, w2t, b1, b2, gamma, beta)


def kernel(h_atom, h_query, edge_index, edge_attr, n_query,
           WQ, WK, WV, Wrbf, W1, b1, W2, b2, ln_gamma, ln_beta):
    e = edge_index.shape[1]
    nq = h_query.shape[0]

    src = edge_index[0].astype(jnp.int32)
    dst = edge_index[1].astype(jnp.int32)
    e_pad = NW * CPW * CH
    pad_e = e_pad - e
    src = jnp.pad(src, (0, pad_e))
    dst = jnp.pad(dst, (0, pad_e), constant_values=TRASH)
    src2d = src.reshape(e_pad // CH, CH)
    dst2d = dst.reshape(e_pad // CH, CH)
    ea_pad = jnp.pad(edge_attr, ((0, pad_e), (0, 0)))
    hq_pad = jnp.pad(h_query, ((0, NQP - nq), (0, 0)))
    ha_pad = jnp.pad(h_atom, ((0, NQP - h_atom.shape[0]), (0, 0)))
    wrbf_pad_t = jnp.pad(Wrbf, ((0, 12), (0, 0))).T

    qq, ka, va = _proj(hq_pad, ha_pad, WQ.T, WK.T, WV.T)
    rbf = _rbf(ea_pad, wrbf_pad_t)

    num_flat, den_flat = _edge_call(qq, ka, va, src2d, dst2d, rbf)

    den0 = jnp.pad(den_flat[:NQD].reshape(NQP, 4)[:nq], ((0, 0), (0, 12)))
    den1 = jnp.pad(den_flat[NQD:].reshape(NQP, 4)[:nq], ((0, 0), (0, 12)))
    out = _mlp(h_query,
               num_flat[:nq], num_flat[NQP:NQP + nq],
               den0, den1,
               W1[:, :HID].T, W1[:, HID:].T, W2.T,
               b1.reshape(1, HID), b2.reshape(1, HID),
               ln_gamma.reshape(1, HID), ln_beta.reshape(1, HID))
    return out


# speedup vs baseline: 1.4846x; 1.0403x over previous
import functools

import jax
import jax.numpy as jnp
from jax import lax
from jax.experimental import pallas as pl
from jax.experimental.pallas import tpu as pltpu
from jax.experimental.pallas import tpu_sc as plsc

N_ATOM = 10000
HID = 128
H = 4
D = HID // H
INV_SQRT_D = 1.0 / (D ** 0.5)

NQP = 10240
TRASH = 10000
NW = 32
CH = 64
NQD = NQP // 32
ROWS_PER_TILE = NQP // 16
DEN_ROWS_PER_TILE = NQD // 8


def _proj_body(hq_ref, ha_ref, wqt_ref, wkt_ref, wvt_ref, qq_ref, ka_ref, va_ref):
    qq_ref[...] = jnp.dot(hq_ref[...], wqt_ref[...], preferred_element_type=jnp.float32)
    ka_ref[...] = jnp.dot(ha_ref[...], wkt_ref[...], preferred_element_type=jnp.float32)
    va_ref[...] = jnp.dot(ha_ref[...], wvt_ref[...], preferred_element_type=jnp.float32)


def _proj(hq_pad, ha_pad, wqt, wkt, wvt):
    n = hq_pad.shape[0]
    blk = 512
    grid = n // blk
    rowspec = pl.BlockSpec((blk, HID), lambda i: (i, 0))
    wspec = pl.BlockSpec((HID, HID), lambda i: (0, 0))
    return pl.pallas_call(
        _proj_body,
        grid=(grid,),
        in_specs=[rowspec, rowspec, wspec, wspec, wspec],
        out_specs=[rowspec, rowspec, rowspec],
        out_shape=[jax.ShapeDtypeStruct((n, HID), jnp.float32)] * 3,
    )(hq_pad, ha_pad, wqt, wkt, wvt)


def _rbf_body(ea_ref, w_ref, out_ref):
    out_ref[...] = jnp.dot(ea_ref[...], w_ref[...], preferred_element_type=jnp.float32)


def _rbf(ea_pad, wrbf_pad_t):
    e = ea_pad.shape[0]
    blk = 2048
    grid = e // blk
    return pl.pallas_call(
        _rbf_body,
        grid=(grid,),
        in_specs=[pl.BlockSpec((blk, 16), lambda i: (i, 0)),
                  pl.BlockSpec((16, 16), lambda i: (0, 0))],
        out_specs=pl.BlockSpec((blk, 16), lambda i: (i, 0)),
        out_shape=jax.ShapeDtypeStruct((e, 16), jnp.float32),
    )(ea_pad, wrbf_pad_t)


def _edge_call(qq, ka, va, src, dst, rbf):
    e_pad = src.shape[0]
    ew = e_pad // NW
    n_chunks = ew // CH
    mesh = plsc.VectorSubcoreMesh(core_axis_name="c", subcore_axis_name="s")

    @functools.partial(
        pl.kernel,
        mesh=mesh,
        compiler_params=pltpu.CompilerParams(needs_layout_passes=False),
        out_type=[jax.ShapeDtypeStruct((2 * NQP, HID), jnp.float32),
                  jax.ShapeDtypeStruct((2 * NQD, HID), jnp.float32)],
        scratch_types=[
            pltpu.VMEM((CH,), jnp.int32),
            pltpu.VMEM((CH,), jnp.int32),
            pltpu.VMEM((CH,), jnp.int32),
            pltpu.VMEM((CH, HID), jnp.float32),
            pltpu.VMEM((CH, HID), jnp.float32),
            pltpu.VMEM((CH, HID), jnp.float32),
            pltpu.VMEM((CH, 16), jnp.float32),
            pltpu.VMEM((CH, HID), jnp.float32),
            pltpu.VMEM_SHARED((NQP, HID), jnp.float32),
            pltpu.VMEM_SHARED((NQD, HID), jnp.float32),
            pltpu.SemaphoreType.DMA,
        ],
    )
    def edge_kernel(qq_hbm, ka_hbm, va_hbm, src_hbm, dst_hbm, rbf_hbm,
                    num_out, den_out,
                    src_v, dst_v, dsh_v, q_rows, k_rows, v_rows, rbf_v, den_src,
                    num_sh, den_sh, gsem):
        cid = lax.axis_index("c")
        sid = lax.axis_index("s")
        wid = cid * 16 + sid
        zeros16 = jnp.zeros((16,), jnp.float32)

        def zero_row(i, _):
            for j in range(HID // 16):
                v_rows[i, pl.ds(j * 16, 16)] = zeros16
                den_src[i, pl.ds(j * 16, 16)] = zeros16
            return 0
        lax.fori_loop(0, CH, zero_row, 0)
        for t in range(ROWS_PER_TILE // CH):
            row0 = sid * ROWS_PER_TILE + t * CH
            pltpu.sync_copy(v_rows, num_sh.at[pl.ds(row0, CH)])
        @pl.when(sid < 8)
        def _():
            pltpu.sync_copy(den_src.at[pl.ds(0, DEN_ROWS_PER_TILE)],
                            den_sh.at[pl.ds(sid * DEN_ROWS_PER_TILE, DEN_ROWS_PER_TILE)])
        plsc.subcore_barrier()

        def chunk_body(c, _):
            base = wid * ew + c * CH
            pltpu.sync_copy(src_hbm.at[pl.ds(base, CH)], src_v)
            pltpu.sync_copy(dst_hbm.at[pl.ds(base, CH)], dst_v)
            pltpu.sync_copy(rbf_hbm.at[pl.ds(base, CH)], rbf_v)
            c1 = pltpu.async_copy(qq_hbm.at[dst_v], q_rows, gsem)
            c2 = pltpu.async_copy(ka_hbm.at[src_v], k_rows, gsem)
            c3 = pltpu.async_copy(va_hbm.at[src_v], v_rows, gsem)
            c1.wait()
            c2.wait()
            c3.wait()

            def group_body(g, _):
                eidx = g * 16 + lax.iota(jnp.int32, 16)
                dm = dst_v[pl.ds(g * 16, 16)]
                dsh_v[pl.ds(g * 16, 16)] = lax.shift_right_logical(dm, 5)
                dlow4 = (dm & 31) * 4
                for h in range(H):
                    hvec = jnp.full((16,), h, jnp.int32)
                    rb = plsc.load_gather(rbf_v, [eidx, hvec])
                    acc = [jnp.zeros((16,), jnp.float32) for _ in range(4)]
                    for j in range(D):
                        col = jnp.full((16,), h * D + j, jnp.int32)
                        qc = plsc.load_gather(q_rows, [eidx, col])
                        kc = plsc.load_gather(k_rows, [eidx, col])
                        acc[j % 4] = acc[j % 4] + qc * kc
                    dot = (acc[0] + acc[1]) + (acc[2] + acc[3])
                    ex = jnp.exp(dot * INV_SQRT_D + rb)
                    plsc.store_scatter(den_src, [eidx, dlow4 + h], ex)
                    for j in range(D):
                        col = jnp.full((16,), h * D + j, jnp.int32)
                        vc = plsc.load_gather(v_rows, [eidx, col])
                        plsc.store_scatter(v_rows, [eidx, col], vc * ex)
                return 0
            lax.fori_loop(0, CH // 16, group_body, 0)

            pltpu.sync_copy(v_rows, num_sh.at[dst_v], add=True)
            pltpu.sync_copy(den_src, den_sh.at[dsh_v], add=True)

            def rezero_body(g, _):
                eidx = g * 16 + lax.iota(jnp.int32, 16)
                dlow4 = (dst_v[pl.ds(g * 16, 16)] & 31) * 4
                for h in range(H):
                    plsc.store_scatter(den_src, [eidx, dlow4 + h], zeros16)
                return 0
            lax.fori_loop(0, CH // 16, rezero_body, 0)
            return 0
        lax.fori_loop(0, n_chunks, chunk_body, 0)

        plsc.subcore_barrier()
        rbase = sid * ROWS_PER_TILE
        pltpu.sync_copy(num_sh.at[pl.ds(rbase, ROWS_PER_TILE)],
                        num_out.at[pl.ds(cid * NQP + rbase, ROWS_PER_TILE)])
        @pl.when(sid < 8)
        def _():
            dbase = sid * DEN_ROWS_PER_TILE
            pltpu.sync_copy(den_sh.at[pl.ds(dbase, DEN_ROWS_PER_TILE)],
                            den_out.at[pl.ds(cid * NQD + dbase, DEN_ROWS_PER_TILE)])

    return edge_kernel(qq, ka, va, src, dst, rbf)


def _mlp_body(hq_ref, n0_ref, n1_ref, d0_ref, d1_ref,
              w1at_ref, w1bt_ref, w2t_ref, b1_ref, b2_ref, g_ref, be_ref,
              out_ref):
    den = d0_ref[...] + d1_ref[...]
    inv = 1.0 / (den + 1e-30)
    num = n0_ref[...] + n1_ref[...]
    r = num.shape[0]
    invw = jnp.concatenate(
        [jnp.broadcast_to(inv[:, h:h + 1], (r, D)) for h in range(H)], axis=1)
    agg = num * invw
    hq = hq_ref[...]
    hdn = jnp.dot(hq, w1at_ref[...], preferred_element_type=jnp.float32)
    hdn = hdn + jnp.dot(agg, w1bt_ref[...], preferred_element_type=jnp.float32)
    hdn = jnp.maximum(hdn + b1_ref[...], 0.0)
    delta = jnp.dot(hdn, w2t_ref[...], preferred_element_type=jnp.float32) + b2_ref[...]
    y = hq + delta
    mu = jnp.mean(y, axis=-1, keepdims=True)
    yc = y - mu
    var = jnp.mean(yc * yc, axis=-1, keepdims=True)
    out_ref[...] = yc * lax.rsqrt(var + 1e-5) * g_ref[...] + be_ref[...]


def _mlp(hq, n0, n1, d0, d1, w1at, w1bt, w2t, b1, b2, gamma, beta):
    n = hq.shape[0]
    blk = 1000
    grid = n // blk
    rowspec = pl.BlockSpec((blk, HID), lambda i: (i, 0))
    denspec = pl.BlockSpec((blk, 16), lambda i: (i, 0))
    wspec = pl.BlockSpec((HID, HID), lambda i: (0, 0))
    vspec = pl.BlockSpec((1, HID), lambda i: (0, 0))
    return pl.pallas_call(
        _mlp_body,
        grid=(grid,),
        in_specs=[rowspec, rowspec, rowspec, denspec, denspec,
                  wspec, wspec, wspec, vspec, vspec, vspec, vspec],
        out_specs=rowspec,
        out_shape=jax.ShapeDtypeStruct((n, HID), jnp.float32),
    )(hq, n0, n1, d0, d1, w1at, w1bt, w2t, b1, b2, gamma, beta)


def kernel(h_atom, h_query, edge_index, edge_attr, n_query,
           WQ, WK, WV, Wrbf, W1, b1, W2, b2, ln_gamma, ln_beta):
    e = edge_index.shape[1]
    nq = h_query.shape[0]

    src = edge_index[0].astype(jnp.int32)
    dst = edge_index[1].astype(jnp.int32)
    e_pad = ((e + NW * CH - 1) // (NW * CH)) * (NW * CH)
    pad_e = e_pad - e
    src = jnp.pad(src, (0, pad_e))
    dst = jnp.pad(dst, (0, pad_e), constant_values=TRASH)
    ea_pad = jnp.pad(edge_attr, ((0, pad_e), (0, 0)))
    hq_pad = jnp.pad(h_query, ((0, NQP - nq), (0, 0)))
    ha_pad = jnp.pad(h_atom, ((0, NQP - h_atom.shape[0]), (0, 0)))
    wrbf_pad_t = jnp.pad(Wrbf, ((0, 12), (0, 0))).T

    qq, ka, va = _proj(hq_pad, ha_pad, WQ.T, WK.T, WV.T)
    rbf = _rbf(ea_pad, wrbf_pad_t)

    num_flat, den_flat = _edge_call(qq, ka, va, src, dst, rbf)

    den0 = jnp.pad(den_flat[:NQD].reshape(NQP, 4)[:nq], ((0, 0), (0, 12)))
    den1 = jnp.pad(den_flat[NQD:].reshape(NQP, 4)[:nq], ((0, 0), (0, 12)))
    out = _mlp(h_query,
               num_flat[:nq], num_flat[NQP:NQP + nq],
               den0, den1,
               W1[:, :HID].T, W1[:, HID:].T, W2.T,
               b1.reshape(1, HID), b2.reshape(1, HID),
               ln_gamma.reshape(1, HID), ln_beta.reshape(1, HID))
    return out
